# baseline (speedup 1.0000x reference)
import dataclasses

import jax
import jax.numpy as jnp
from jax import lax
from jax.experimental import pallas as pl
from jax.experimental.pallas import tpu as pltpu
from jax.experimental.pallas import tpu_sc as plsc

BATCH = 4096
SEQ = 200
DIM = 64
LANES = 16
NCORES = 2
NSUB = 16
NWORKERS = NCORES * NSUB
RPW = BATCH // NWORKERS
CA = 104
CB = SEQ - CA
NCLS = 5
NEG_HUGE = -3.4e38


def _sc_body(x_hbm, table_hbm, wt_hbm, bias_hbm, outf_hbm, pred_hbm,
             idx_v, rows_v, wt_v, bias_v, outf_v, pred_v):
    wid = lax.axis_index("s") * NCORES + lax.axis_index("c")
    base = pl.multiple_of(wid * (RPW * SEQ), 8)

    pltpu.sync_copy(x_hbm.at[pl.ds(base, RPW * SEQ)], idx_v)
    pltpu.sync_copy(wt_hbm, wt_v)
    pltpu.sync_copy(bias_hbm, bias_v)

    lane = lax.iota(jnp.int32, LANES)
    cls_mask = lane < NCLS

    @pl.loop(0, RPW)
    def _row(r):
        off = pl.multiple_of(r * SEQ, 8)
        pltpu.sync_copy(
            table_hbm.at[idx_v.at[pl.ds(off, CA)]], rows_v.at[pl.ds(0, CA)])
        pltpu.sync_copy(
            table_hbm.at[idx_v.at[pl.ds(off + CA, CB)]],
            rows_v.at[pl.ds(CA, CB)])

        def acc_body(s, acc):
            a0, a1, a2, a3 = acc
            a0 = a0 + rows_v[s, pl.ds(0 * LANES, LANES)]
            a1 = a1 + rows_v[s, pl.ds(1 * LANES, LANES)]
            a2 = a2 + rows_v[s, pl.ds(2 * LANES, LANES)]
            a3 = a3 + rows_v[s, pl.ds(3 * LANES, LANES)]
            return (a0, a1, a2, a3)

        zero = jnp.zeros((LANES,), jnp.float32)
        a0, a1, a2, a3 = lax.fori_loop(
            0, SEQ, acc_body, (zero, zero, zero, zero), unroll=4)

        out16 = bias_v[...]
        for k, a in enumerate((a0, a1, a2, a3)):
            for j in range(LANES):
                out16 = out16 + a[j] * wt_v[k * LANES + j, :]
        outf_v[r, :] = out16

        masked = jnp.where(cls_mask, out16, NEG_HUGE)
        m = jnp.max(masked)
        pred_v[r, :] = plsc.all_reduce_ffs(masked == m).astype(jnp.int32)

    obase = pl.multiple_of(wid * RPW, 8)
    pltpu.sync_copy(outf_v, outf_hbm.at[pl.ds(obase, RPW)])
    pltpu.sync_copy(pred_v, pred_hbm.at[pl.ds(obase, RPW)])


def kernel(x, table, W, b):
    x_flat = x.reshape(-1).astype(jnp.int32)
    wt16 = jnp.zeros((DIM, LANES), jnp.float32).at[:, :NCLS].set(W.T)
    bias16 = jnp.zeros((LANES,), jnp.float32).at[:NCLS].set(b * float(SEQ))

    cp = pltpu.CompilerParams()
    if "needs_layout_passes" in pltpu.CompilerParams.__dataclass_fields__:
        cp = dataclasses.replace(cp, needs_layout_passes=False)
    if "use_tc_tiling_on_sc" in pltpu.CompilerParams.__dataclass_fields__:
        cp = dataclasses.replace(cp, use_tc_tiling_on_sc=False)
    mesh = plsc.VectorSubcoreMesh(core_axis_name="c", subcore_axis_name="s")
    call = pl.kernel(
        _sc_body,
        out_type=(
            jax.ShapeDtypeStruct((BATCH, LANES), jnp.float32),
            jax.ShapeDtypeStruct((BATCH, LANES), jnp.int32),
        ),
        mesh=mesh,
        scratch_types=[
            pltpu.VMEM((RPW * SEQ,), jnp.int32),
            pltpu.VMEM((SEQ, DIM), jnp.float32),
            pltpu.VMEM((DIM, LANES), jnp.float32),
            pltpu.VMEM((LANES,), jnp.float32),
            pltpu.VMEM((RPW, LANES), jnp.float32),
            pltpu.VMEM((RPW, LANES), jnp.int32),
        ],
        compiler_params=cp,
    )
    outf, pred = call(x_flat, table, wt16, bias16)
    return (outf[:, :NCLS], pred[:, 0])


# speedup vs baseline: 1.3096x; 1.3096x over previous
import dataclasses

import jax
import jax.numpy as jnp
from jax import lax
from jax.experimental import pallas as pl
from jax.experimental.pallas import tpu as pltpu
from jax.experimental.pallas import tpu_sc as plsc

BATCH = 4096
SEQ = 200
DIM = 64
LANES = 16
NCORES = 2
NSUB = 16
NWORKERS = NCORES * NSUB
RPW = BATCH // NWORKERS
CA = 104
CB = SEQ - CA
NCLS = 5
NEG_HUGE = -3.4e38
NBUF = 4


def _sc_body(x_hbm, table_hbm, wt_hbm, bias_hbm, outf_hbm, pred_hbm,
             idx_v, bufs_v, wt_v, bias_v, outf_v, pred_v, *sems):
    wid = lax.axis_index("s") * NCORES + lax.axis_index("c")
    base = pl.multiple_of(wid * (RPW * SEQ), 8)

    pltpu.sync_copy(x_hbm.at[pl.ds(base, RPW * SEQ)], idx_v)
    pltpu.sync_copy(wt_hbm, wt_v)
    pltpu.sync_copy(bias_hbm, bias_v)

    lane = lax.iota(jnp.int32, LANES)
    cls_mask = lane < NCLS

    def issue(r, b):
        off = pl.multiple_of(r * SEQ, 8)
        pltpu.async_copy(table_hbm.at[idx_v.at[pl.ds(off, CA)]],
                         bufs_v.at[b].at[pl.ds(0, CA)], sems[b])
        pltpu.async_copy(table_hbm.at[idx_v.at[pl.ds(off + CA, CB)]],
                         bufs_v.at[b].at[pl.ds(CA, CB)], sems[b])

    def drain(b):
        pltpu.make_async_copy(table_hbm.at[pl.ds(0, SEQ)],
                              bufs_v.at[b], sems[b]).wait()

    for rr in range(NBUF - 1):
        issue(rr, rr)

    @pl.loop(0, RPW, step=NBUF)
    def _rows(rbase):
        for db in range(NBUF):
            r = rbase + db
            rpre = r + NBUF - 1

            @pl.when(rpre < RPW)
            def _():
                issue(rpre, (db + NBUF - 1) % NBUF)

            drain(db)
            rows = bufs_v.at[db]

            def acc_body(s, acc):
                a0, a1, a2, a3 = acc
                a0 = a0 + rows[s, pl.ds(0 * LANES, LANES)]
                a1 = a1 + rows[s, pl.ds(1 * LANES, LANES)]
                a2 = a2 + rows[s, pl.ds(2 * LANES, LANES)]
                a3 = a3 + rows[s, pl.ds(3 * LANES, LANES)]
                return (a0, a1, a2, a3)

            zero = jnp.zeros((LANES,), jnp.float32)
            a0, a1, a2, a3 = lax.fori_loop(
                0, SEQ, acc_body, (zero, zero, zero, zero), unroll=8)

            out16 = bias_v[...]
            for k, a in enumerate((a0, a1, a2, a3)):
                for j in range(LANES):
                    out16 = out16 + a[j] * wt_v[k * LANES + j, :]
            outf_v[r, :] = out16

            masked = jnp.where(cls_mask, out16, NEG_HUGE)
            m = jnp.max(masked)
            pred_v[r, :] = plsc.all_reduce_ffs(masked == m).astype(jnp.int32)

    obase = pl.multiple_of(wid * RPW, 8)
    pltpu.sync_copy(outf_v, outf_hbm.at[pl.ds(obase, RPW)])
    pltpu.sync_copy(pred_v, pred_hbm.at[pl.ds(obase, RPW)])


def kernel(x, table, W, b):
    x_flat = x.reshape(-1).astype(jnp.int32)
    wt16 = jnp.zeros((DIM, LANES), jnp.float32).at[:, :NCLS].set(W.T)
    bias16 = jnp.zeros((LANES,), jnp.float32).at[:NCLS].set(b * float(SEQ))

    cp = pltpu.CompilerParams()
    if "needs_layout_passes" in pltpu.CompilerParams.__dataclass_fields__:
        cp = dataclasses.replace(cp, needs_layout_passes=False)
    if "use_tc_tiling_on_sc" in pltpu.CompilerParams.__dataclass_fields__:
        cp = dataclasses.replace(cp, use_tc_tiling_on_sc=False)
    mesh = plsc.VectorSubcoreMesh(core_axis_name="c", subcore_axis_name="s")
    call = pl.kernel(
        _sc_body,
        out_type=(
            jax.ShapeDtypeStruct((BATCH, LANES), jnp.float32),
            jax.ShapeDtypeStruct((BATCH, LANES), jnp.int32),
        ),
        mesh=mesh,
        scratch_types=[
            pltpu.VMEM((RPW * SEQ,), jnp.int32),
            pltpu.VMEM((NBUF, SEQ, DIM), jnp.float32),
            pltpu.VMEM((DIM, LANES), jnp.float32),
            pltpu.VMEM((LANES,), jnp.float32),
            pltpu.VMEM((RPW, LANES), jnp.float32),
            pltpu.VMEM((RPW, LANES), jnp.int32),
        ] + [pltpu.SemaphoreType.DMA] * NBUF,
        compiler_params=cp,
    )
    outf, pred = call(x_flat, table, wt16, bias16)
    return (outf[:, :NCLS], pred[:, 0])
